# baseline (speedup 1.0000x reference)
import functools

import jax
import jax.numpy as jnp
from jax import lax
from jax.experimental import pallas as pl
from jax.experimental.pallas import tpu as pltpu
from jax.experimental.pallas import tpu_sc as plsc

_N = 1_000_000
_D = 32
_R = 512
_NW = 32
_CH = 1953
_NSC = _CH * _R
_NK = _CH // _NW
_EXTRA = _CH - _NK * _NW


def _compute_chunk(ub, vb, ob):
    def group(g, carry):
        l0 = g * 16
        ps = [ub[j, pl.ds(l0, 16)] * vb[j, pl.ds(l0, 16)] for j in range(_D)]
        while len(ps) > 1:
            ps = [ps[i] + ps[i + 1] for i in range(0, len(ps), 2)]
        ob[pl.ds(l0, 16)] = ps[0]
        return carry

    lax.fori_loop(0, _R // 16, group, 0)


def _sc_rowdot(u_hbm, v_hbm, o_hbm, u0, u1, v0, v1, o0, o1,
               si0, si1, so0, so1):
    wid = lax.axis_index("s") * 2 + lax.axis_index("c")

    def start_in(c, ub, vb, sem):
        pltpu.async_copy(u_hbm.at[:, pl.ds(c * _R, _R)], ub, sem)
        pltpu.async_copy(v_hbm.at[:, pl.ds(c * _R, _R)], vb, sem)

    def wait_in(c, ub, vb, sem):
        pltpu.make_async_copy(u_hbm.at[:, pl.ds(c * _R, _R)], ub, sem).wait()
        pltpu.make_async_copy(v_hbm.at[:, pl.ds(c * _R, _R)], vb, sem).wait()

    def process(k, ub, vb, ob, sin, sout, ub_n, vb_n, sin_n):
        c = wid + k * _NW

        @pl.when(k + 1 < _NK)
        def _prefetch():
            start_in(c + _NW, ub_n, vb_n, sin_n)

        wait_in(c, ub, vb, sin)
        _compute_chunk(ub, vb, ob)

        @pl.when(k >= 2)
        def _drain_prev_out():
            prev = (c - 2 * _NW) * _R
            pltpu.make_async_copy(ob, o_hbm.at[pl.ds(prev, _R)], sout).wait()

        pltpu.async_copy(ob, o_hbm.at[pl.ds(c * _R, _R)], sout)

    start_in(wid, u0, v0, si0)

    def round_body(k, carry):
        @pl.when(k % 2 == 0)
        def _even():
            process(k, u0, v0, o0, si0, so0, u1, v1, si1)

        @pl.when(k % 2 == 1)
        def _odd():
            process(k, u1, v1, o1, si1, so1, u0, v0, si0)

        return carry

    lax.fori_loop(0, _NK, round_body, 0)

    pltpu.make_async_copy(o0, o_hbm.at[pl.ds(wid * _R, _R)], so0).wait()
    pltpu.make_async_copy(o1, o_hbm.at[pl.ds(wid * _R, _R)], so1).wait()

    @pl.when(wid < _EXTRA)
    def _tail():
        c = _NK * _NW + wid
        pltpu.sync_copy(u_hbm.at[:, pl.ds(c * _R, _R)], u0)
        pltpu.sync_copy(v_hbm.at[:, pl.ds(c * _R, _R)], v0)
        _compute_chunk(u0, v0, o0)
        pltpu.sync_copy(o0, o_hbm.at[pl.ds(c * _R, _R)])


def _tc_body(u_ref, v_ref, o_ref):
    o_ref[...] = jnp.sum(u_ref[...] * v_ref[...], axis=0)


def kernel(user_emb, items_emb):
    n, d = user_emb.shape
    ut = user_emb.T
    vt = items_emb.T

    mesh = plsc.VectorSubcoreMesh(core_axis_name="c", subcore_axis_name="s")
    sc_run = functools.partial(
        pl.kernel,
        mesh=mesh,
        compiler_params=pltpu.CompilerParams(use_tc_tiling_on_sc=True),
        out_type=jax.ShapeDtypeStruct((_NSC,), jnp.float32),
        scratch_types=[
            pltpu.VMEM((_D, _R), jnp.float32),
            pltpu.VMEM((_D, _R), jnp.float32),
            pltpu.VMEM((_D, _R), jnp.float32),
            pltpu.VMEM((_D, _R), jnp.float32),
            pltpu.VMEM((_R,), jnp.float32),
            pltpu.VMEM((_R,), jnp.float32),
            pltpu.SemaphoreType.DMA,
            pltpu.SemaphoreType.DMA,
            pltpu.SemaphoreType.DMA,
            pltpu.SemaphoreType.DMA,
        ],
    )(_sc_rowdot)
    sc_out = sc_run(ut, vt)

    n_tail = n - _NSC
    ut_tail = lax.slice(ut, (0, _NSC), (d, n))
    vt_tail = lax.slice(vt, (0, _NSC), (d, n))
    tc_out = pl.pallas_call(
        _tc_body,
        grid=(1,),
        in_specs=[
            pl.BlockSpec((d, n_tail), lambda i: (0, 0)),
            pl.BlockSpec((d, n_tail), lambda i: (0, 0)),
        ],
        out_specs=pl.BlockSpec((n_tail,), lambda i: (0,)),
        out_shape=jax.ShapeDtypeStruct((n_tail,), jnp.float32),
    )(ut_tail, vt_tail)

    return jnp.concatenate([sc_out, tc_out])


# speedup vs baseline: 1.3939x; 1.3939x over previous
import functools

import jax
import jax.numpy as jnp
from jax import lax
from jax.experimental import pallas as pl
from jax.experimental.pallas import tpu as pltpu
from jax.experimental.pallas import tpu_sc as plsc

_N = 1_000_000
_D = 32
_R = 512
_NW = 32
_NK = 24
_S = _R * _NW * _NK
_TCB = 65536


def _compute_chunk(ub, vb, ob):
    def group(g, carry):
        l0 = g * 16
        ps = [ub[j, pl.ds(l0, 16)] * vb[j, pl.ds(l0, 16)] for j in range(_D)]
        while len(ps) > 1:
            ps = [ps[i] + ps[i + 1] for i in range(0, len(ps), 2)]
        ob[pl.ds(l0, 16)] = ps[0]
        return carry

    lax.fori_loop(0, _R // 16, group, 0)


def _sc_rowdot(u_hbm, v_hbm, o_hbm, u0, u1, v0, v1, o0, o1,
               si0, si1, so0, so1):
    wid = lax.axis_index("s") * 2 + lax.axis_index("c")

    def start_in(c, ub, vb, sem):
        pltpu.async_copy(u_hbm.at[:, pl.ds(c * _R, _R)], ub, sem)
        pltpu.async_copy(v_hbm.at[:, pl.ds(c * _R, _R)], vb, sem)

    def wait_in(c, ub, vb, sem):
        pltpu.make_async_copy(u_hbm.at[:, pl.ds(c * _R, _R)], ub, sem).wait()
        pltpu.make_async_copy(v_hbm.at[:, pl.ds(c * _R, _R)], vb, sem).wait()

    def process(k, ub, vb, ob, sin, sout, ub_n, vb_n, sin_n):
        c = wid + k * _NW

        @pl.when(k + 1 < _NK)
        def _prefetch():
            start_in(c + _NW, ub_n, vb_n, sin_n)

        wait_in(c, ub, vb, sin)
        _compute_chunk(ub, vb, ob)

        @pl.when(k >= 2)
        def _drain_prev_out():
            prev = (c - 2 * _NW) * _R
            pltpu.make_async_copy(ob, o_hbm.at[pl.ds(prev, _R)], sout).wait()

        pltpu.async_copy(ob, o_hbm.at[pl.ds(c * _R, _R)], sout)

    start_in(wid, u0, v0, si0)

    def round_body(k, carry):
        @pl.when(k % 2 == 0)
        def _even():
            process(k, u0, v0, o0, si0, so0, u1, v1, si1)

        @pl.when(k % 2 == 1)
        def _odd():
            process(k, u1, v1, o1, si1, so1, u0, v0, si0)

        return carry

    lax.fori_loop(0, _NK, round_body, 0)

    pltpu.make_async_copy(o0, o_hbm.at[pl.ds(wid * _R, _R)], so0).wait()
    pltpu.make_async_copy(o1, o_hbm.at[pl.ds(wid * _R, _R)], so1).wait()


def _tc_body(u_ref, v_ref, o_ref):
    o_ref[...] = jnp.sum(u_ref[...] * v_ref[...], axis=0)


def kernel(user_emb, items_emb):
    n, d = user_emb.shape
    ut = user_emb.T
    vt = items_emb.T

    mesh = plsc.VectorSubcoreMesh(core_axis_name="c", subcore_axis_name="s")
    sc_run = functools.partial(
        pl.kernel,
        mesh=mesh,
        compiler_params=pltpu.CompilerParams(use_tc_tiling_on_sc=True),
        out_type=jax.ShapeDtypeStruct((_S,), jnp.float32),
        scratch_types=[
            pltpu.VMEM((_D, _R), jnp.float32),
            pltpu.VMEM((_D, _R), jnp.float32),
            pltpu.VMEM((_D, _R), jnp.float32),
            pltpu.VMEM((_D, _R), jnp.float32),
            pltpu.VMEM((_R,), jnp.float32),
            pltpu.VMEM((_R,), jnp.float32),
            pltpu.SemaphoreType.DMA,
            pltpu.SemaphoreType.DMA,
            pltpu.SemaphoreType.DMA,
            pltpu.SemaphoreType.DMA,
        ],
    )(_sc_rowdot)
    sc_out = sc_run(ut, vt)

    n_tc = n - _S
    blk0 = _S // _TCB
    tc_out = pl.pallas_call(
        _tc_body,
        grid=(pl.cdiv(n_tc, _TCB),),
        in_specs=[
            pl.BlockSpec((d, _TCB), lambda i: (0, blk0 + i)),
            pl.BlockSpec((d, _TCB), lambda i: (0, blk0 + i)),
        ],
        out_specs=pl.BlockSpec((_TCB,), lambda i: (i,)),
        out_shape=jax.ShapeDtypeStruct((n_tc,), jnp.float32),
    )(ut, vt)

    return jnp.concatenate([sc_out, tc_out])


# speedup vs baseline: 1.4157x; 1.0157x over previous
import functools

import jax
import jax.numpy as jnp
from jax import lax
from jax.experimental import pallas as pl
from jax.experimental.pallas import tpu as pltpu
from jax.experimental.pallas import tpu_sc as plsc

_N = 1_000_000
_D = 32
_R = 512
_NW = 32
_NK = 12
_S = _R * _NW * _NK
_TCB = 65536


def _compute_chunk(ub, vb, ob):
    def group(g, carry):
        l0 = g * 16
        ps = [ub[j, pl.ds(l0, 16)] * vb[j, pl.ds(l0, 16)] for j in range(_D)]
        while len(ps) > 1:
            ps = [ps[i] + ps[i + 1] for i in range(0, len(ps), 2)]
        ob[pl.ds(l0, 16)] = ps[0]
        return carry

    lax.fori_loop(0, _R // 16, group, 0)


def _sc_rowdot(u_hbm, v_hbm, o_hbm, u0, u1, v0, v1, o0, o1,
               si0, si1, so0, so1):
    wid = lax.axis_index("s") * 2 + lax.axis_index("c")

    def start_in(c, ub, vb, sem):
        pltpu.async_copy(u_hbm.at[:, pl.ds(c * _R, _R)], ub, sem)
        pltpu.async_copy(v_hbm.at[:, pl.ds(c * _R, _R)], vb, sem)

    def wait_in(c, ub, vb, sem):
        pltpu.make_async_copy(u_hbm.at[:, pl.ds(c * _R, _R)], ub, sem).wait()
        pltpu.make_async_copy(v_hbm.at[:, pl.ds(c * _R, _R)], vb, sem).wait()

    def process(k, ub, vb, ob, sin, sout, ub_n, vb_n, sin_n):
        c = wid + k * _NW

        @pl.when(k + 1 < _NK)
        def _prefetch():
            start_in(c + _NW, ub_n, vb_n, sin_n)

        wait_in(c, ub, vb, sin)
        _compute_chunk(ub, vb, ob)

        @pl.when(k >= 2)
        def _drain_prev_out():
            prev = (c - 2 * _NW) * _R
            pltpu.make_async_copy(ob, o_hbm.at[pl.ds(prev, _R)], sout).wait()

        pltpu.async_copy(ob, o_hbm.at[pl.ds(c * _R, _R)], sout)

    start_in(wid, u0, v0, si0)

    def round_body(k, carry):
        @pl.when(k % 2 == 0)
        def _even():
            process(k, u0, v0, o0, si0, so0, u1, v1, si1)

        @pl.when(k % 2 == 1)
        def _odd():
            process(k, u1, v1, o1, si1, so1, u0, v0, si0)

        return carry

    lax.fori_loop(0, _NK, round_body, 0)

    pltpu.make_async_copy(o0, o_hbm.at[pl.ds(wid * _R, _R)], so0).wait()
    pltpu.make_async_copy(o1, o_hbm.at[pl.ds(wid * _R, _R)], so1).wait()


def _tc_body(u_ref, v_ref, o_ref):
    o_ref[...] = jnp.sum(u_ref[...] * v_ref[...], axis=0)


def kernel(user_emb, items_emb):
    n, d = user_emb.shape
    ut = user_emb.T
    vt = items_emb.T

    mesh = plsc.VectorSubcoreMesh(core_axis_name="c", subcore_axis_name="s")
    sc_run = functools.partial(
        pl.kernel,
        mesh=mesh,
        compiler_params=pltpu.CompilerParams(use_tc_tiling_on_sc=True),
        out_type=jax.ShapeDtypeStruct((_S,), jnp.float32),
        scratch_types=[
            pltpu.VMEM((_D, _R), jnp.float32),
            pltpu.VMEM((_D, _R), jnp.float32),
            pltpu.VMEM((_D, _R), jnp.float32),
            pltpu.VMEM((_D, _R), jnp.float32),
            pltpu.VMEM((_R,), jnp.float32),
            pltpu.VMEM((_R,), jnp.float32),
            pltpu.SemaphoreType.DMA,
            pltpu.SemaphoreType.DMA,
            pltpu.SemaphoreType.DMA,
            pltpu.SemaphoreType.DMA,
        ],
    )(_sc_rowdot)
    sc_out = sc_run(ut, vt)

    n_tc = n - _S
    blk0 = _S // _TCB
    tc_out = pl.pallas_call(
        _tc_body,
        grid=(pl.cdiv(n_tc, _TCB),),
        in_specs=[
            pl.BlockSpec((d, _TCB), lambda i: (0, blk0 + i)),
            pl.BlockSpec((d, _TCB), lambda i: (0, blk0 + i)),
        ],
        out_specs=pl.BlockSpec((_TCB,), lambda i: (i,)),
        out_shape=jax.ShapeDtypeStruct((n_tc,), jnp.float32),
    )(ut, vt)

    return jnp.concatenate([sc_out, tc_out])


# speedup vs baseline: 1.4304x; 1.0104x over previous
import functools

import jax
import jax.numpy as jnp
from jax import lax
from jax.experimental import pallas as pl
from jax.experimental.pallas import tpu as pltpu
from jax.experimental.pallas import tpu_sc as plsc

_N = 1_000_000
_D = 32
_R = 512
_NW = 32
_NK = 4
_S = _R * _NW * _NK
_TCB = 65536


def _compute_chunk(ub, vb, ob):
    def group(g, carry):
        l0 = g * 16
        ps = [ub[j, pl.ds(l0, 16)] * vb[j, pl.ds(l0, 16)] for j in range(_D)]
        while len(ps) > 1:
            ps = [ps[i] + ps[i + 1] for i in range(0, len(ps), 2)]
        ob[pl.ds(l0, 16)] = ps[0]
        return carry

    lax.fori_loop(0, _R // 16, group, 0)


def _sc_rowdot(u_hbm, v_hbm, o_hbm, u0, u1, v0, v1, o0, o1,
               si0, si1, so0, so1):
    wid = lax.axis_index("s") * 2 + lax.axis_index("c")

    def start_in(c, ub, vb, sem):
        pltpu.async_copy(u_hbm.at[:, pl.ds(c * _R, _R)], ub, sem)
        pltpu.async_copy(v_hbm.at[:, pl.ds(c * _R, _R)], vb, sem)

    def wait_in(c, ub, vb, sem):
        pltpu.make_async_copy(u_hbm.at[:, pl.ds(c * _R, _R)], ub, sem).wait()
        pltpu.make_async_copy(v_hbm.at[:, pl.ds(c * _R, _R)], vb, sem).wait()

    def process(k, ub, vb, ob, sin, sout, ub_n, vb_n, sin_n):
        c = wid + k * _NW

        @pl.when(k + 1 < _NK)
        def _prefetch():
            start_in(c + _NW, ub_n, vb_n, sin_n)

        wait_in(c, ub, vb, sin)
        _compute_chunk(ub, vb, ob)

        @pl.when(k >= 2)
        def _drain_prev_out():
            prev = (c - 2 * _NW) * _R
            pltpu.make_async_copy(ob, o_hbm.at[pl.ds(prev, _R)], sout).wait()

        pltpu.async_copy(ob, o_hbm.at[pl.ds(c * _R, _R)], sout)

    start_in(wid, u0, v0, si0)

    def round_body(k, carry):
        @pl.when(k % 2 == 0)
        def _even():
            process(k, u0, v0, o0, si0, so0, u1, v1, si1)

        @pl.when(k % 2 == 1)
        def _odd():
            process(k, u1, v1, o1, si1, so1, u0, v0, si0)

        return carry

    lax.fori_loop(0, _NK, round_body, 0)

    pltpu.make_async_copy(o0, o_hbm.at[pl.ds(wid * _R, _R)], so0).wait()
    pltpu.make_async_copy(o1, o_hbm.at[pl.ds(wid * _R, _R)], so1).wait()


def _tc_body(u_ref, v_ref, o_ref):
    o_ref[...] = jnp.sum(u_ref[...] * v_ref[...], axis=0)


def kernel(user_emb, items_emb):
    n, d = user_emb.shape
    ut = user_emb.T
    vt = items_emb.T

    mesh = plsc.VectorSubcoreMesh(core_axis_name="c", subcore_axis_name="s")
    sc_run = functools.partial(
        pl.kernel,
        mesh=mesh,
        compiler_params=pltpu.CompilerParams(use_tc_tiling_on_sc=True),
        out_type=jax.ShapeDtypeStruct((_S,), jnp.float32),
        scratch_types=[
            pltpu.VMEM((_D, _R), jnp.float32),
            pltpu.VMEM((_D, _R), jnp.float32),
            pltpu.VMEM((_D, _R), jnp.float32),
            pltpu.VMEM((_D, _R), jnp.float32),
            pltpu.VMEM((_R,), jnp.float32),
            pltpu.VMEM((_R,), jnp.float32),
            pltpu.SemaphoreType.DMA,
            pltpu.SemaphoreType.DMA,
            pltpu.SemaphoreType.DMA,
            pltpu.SemaphoreType.DMA,
        ],
    )(_sc_rowdot)
    sc_out = sc_run(ut, vt)

    n_tc = n - _S
    blk0 = _S // _TCB
    tc_out = pl.pallas_call(
        _tc_body,
        grid=(pl.cdiv(n_tc, _TCB),),
        in_specs=[
            pl.BlockSpec((d, _TCB), lambda i: (0, blk0 + i)),
            pl.BlockSpec((d, _TCB), lambda i: (0, blk0 + i)),
        ],
        out_specs=pl.BlockSpec((_TCB,), lambda i: (i,)),
        out_shape=jax.ShapeDtypeStruct((n_tc,), jnp.float32),
    )(ut, vt)

    return jnp.concatenate([sc_out, tc_out])


# speedup vs baseline: 1.4306x; 1.0001x over previous
import functools

import jax
import jax.numpy as jnp
from jax import lax
from jax.experimental import pallas as pl
from jax.experimental.pallas import tpu as pltpu
from jax.experimental.pallas import tpu_sc as plsc

_N = 1_000_000
_D = 32
_R = 512
_NW = 32
_NK = 4
_S = _R * _NW * _NK
_TCB = 65536


def _compute_chunk(ub, vb, ob):
    def group(g, carry):
        l0 = g * 16
        ps = [ub[j, pl.ds(l0, 16)] * vb[j, pl.ds(l0, 16)] for j in range(_D)]
        while len(ps) > 1:
            ps = [ps[i] + ps[i + 1] for i in range(0, len(ps), 2)]
        ob[pl.ds(l0, 16)] = ps[0]
        return carry

    lax.fori_loop(0, _R // 16, group, 0)


def _sc_rowdot(u_hbm, v_hbm, o_hbm, u0, u1, v0, v1, o0, o1,
               si0, si1, so0, so1):
    wid = lax.axis_index("s") * 2 + lax.axis_index("c")

    def start_in(c, ub, vb, sem):
        pltpu.async_copy(u_hbm.at[:, pl.ds(c * _R, _R)], ub, sem)
        pltpu.async_copy(v_hbm.at[:, pl.ds(c * _R, _R)], vb, sem)

    def wait_in(c, ub, vb, sem):
        pltpu.make_async_copy(u_hbm.at[:, pl.ds(c * _R, _R)], ub, sem).wait()
        pltpu.make_async_copy(v_hbm.at[:, pl.ds(c * _R, _R)], vb, sem).wait()

    def process(k, ub, vb, ob, sin, sout, ub_n, vb_n, sin_n):
        c = wid + k * _NW

        @pl.when(k + 1 < _NK)
        def _prefetch():
            start_in(c + _NW, ub_n, vb_n, sin_n)

        wait_in(c, ub, vb, sin)
        _compute_chunk(ub, vb, ob)

        @pl.when(k >= 2)
        def _drain_prev_out():
            prev = (c - 2 * _NW) * _R
            pltpu.make_async_copy(ob, o_hbm.at[pl.ds(prev, _R)], sout).wait()

        pltpu.async_copy(ob, o_hbm.at[pl.ds(c * _R, _R)], sout)

    start_in(wid, u0, v0, si0)

    def round_body(k, carry):
        @pl.when(k % 2 == 0)
        def _even():
            process(k, u0, v0, o0, si0, so0, u1, v1, si1)

        @pl.when(k % 2 == 1)
        def _odd():
            process(k, u1, v1, o1, si1, so1, u0, v0, si0)

        return carry

    lax.fori_loop(0, _NK, round_body, 0)

    pltpu.make_async_copy(o0, o_hbm.at[pl.ds(wid * _R, _R)], so0).wait()
    pltpu.make_async_copy(o1, o_hbm.at[pl.ds(wid * _R, _R)], so1).wait()


def _tc_body(u_ref, v_ref, o_ref):
    o_ref[...] = jnp.sum(u_ref[...] * v_ref[...], axis=0)


def kernel(user_emb, items_emb):
    n, d = user_emb.shape
    ut = user_emb.T
    vt = items_emb.T

    mesh = plsc.VectorSubcoreMesh(core_axis_name="c", subcore_axis_name="s")
    sc_run = functools.partial(
        pl.kernel,
        mesh=mesh,
        compiler_params=pltpu.CompilerParams(
            use_tc_tiling_on_sc=True, skip_device_barrier=True),
        out_type=jax.ShapeDtypeStruct((_S,), jnp.float32),
        scratch_types=[
            pltpu.VMEM((_D, _R), jnp.float32),
            pltpu.VMEM((_D, _R), jnp.float32),
            pltpu.VMEM((_D, _R), jnp.float32),
            pltpu.VMEM((_D, _R), jnp.float32),
            pltpu.VMEM((_R,), jnp.float32),
            pltpu.VMEM((_R,), jnp.float32),
            pltpu.SemaphoreType.DMA,
            pltpu.SemaphoreType.DMA,
            pltpu.SemaphoreType.DMA,
            pltpu.SemaphoreType.DMA,
        ],
    )(_sc_rowdot)
    sc_out = sc_run(ut, vt)

    n_tc = n - _S
    blk0 = _S // _TCB
    tc_out = pl.pallas_call(
        _tc_body,
        grid=(pl.cdiv(n_tc, _TCB),),
        in_specs=[
            pl.BlockSpec((d, _TCB), lambda i: (0, blk0 + i)),
            pl.BlockSpec((d, _TCB), lambda i: (0, blk0 + i)),
        ],
        out_specs=pl.BlockSpec((_TCB,), lambda i: (i,)),
        out_shape=jax.ShapeDtypeStruct((n_tc,), jnp.float32),
    )(ut, vt)

    return jnp.concatenate([sc_out, tc_out])
